# Initial kernel scaffold; baseline (speedup 1.0000x reference)
import jax
import jax.numpy as jnp
from jax.experimental import pallas as pl


def kernel(x, edge_index, confidence_maps, w_enc, bvec, w_sp1, w_sp2, w_gat, u_src, u_dst, b_gat, w_out, b_out):
    raise NotImplementedError("write your pallas kernel here")


# speedup vs baseline: 1.2293x; 1.2293x over previous
import functools

import jax
import jax.numpy as jnp
from jax.experimental import pallas as pl
from jax.experimental.pallas import tpu as pltpu


def _spatial_body(x_ref, conf_ref, wenc_ref, bvec_ref, w1_ref, w2_ref,
                  hout_ref, pooled_ref, *, H, W):
    HW = H * W
    hid = wenc_ref.shape[0]

    x = x_ref[0].astype(jnp.bfloat16)
    conf = conf_ref[0]

    h = jnp.dot(wenc_ref[...], x, preferred_element_type=jnp.float32)
    h = jnp.maximum(h + bvec_ref[0], 0.0) * conf

    p = jax.lax.broadcasted_iota(jnp.int32, (1, HW), 1)
    xcol = p % W
    yrow = p // W
    m_xm = xcol >= 1
    m_xp = xcol <= W - 2
    m_ym = yrow >= 1
    m_yp = yrow <= H - 2

    def conv3x3(v, w_ref, bias):
        zero = jnp.zeros((), jnp.bfloat16)
        vxm = jnp.where(m_xm, pltpu.roll(v, 1, axis=1), zero)
        vxp = jnp.where(m_xp, pltpu.roll(v, HW - 1, axis=1), zero)
        c3 = jnp.concatenate([vxm, v, vxp], axis=0)
        up = jnp.where(m_ym, pltpu.roll(c3, W, axis=1), zero)
        dn = jnp.where(m_yp, pltpu.roll(c3, HW - W, axis=1), zero)
        u = jnp.concatenate([up, c3, dn], axis=0)
        g = jnp.dot(w_ref[...], u, preferred_element_type=jnp.float32)
        return jnp.maximum(g + bias, 0.0)

    h1 = conv3x3(h.astype(jnp.bfloat16), w1_ref, bvec_ref[1])
    h2 = conv3x3(h1.astype(jnp.bfloat16), w2_ref, bvec_ref[2])

    hout_ref[...] = h2.astype(jnp.bfloat16).reshape(1, hid, HW)

    inv_hw = jnp.full((1, HW), 1.0 / HW, jnp.float32)
    pooled = jax.lax.dot_general(inv_hw, h2, (((1,), (1,)), ((), ())),
                                 preferred_element_type=jnp.float32)
    pooled_ref[...] = pooled.reshape(1, 1, hid)


def _run_spatial(x, conf, wenc_bf, bvec, w1_cat, w2_cat, H, W):
    N, C, HW = x.shape
    hid = wenc_bf.shape[0]
    body = functools.partial(_spatial_body, H=H, W=W)
    return pl.pallas_call(
        body,
        out_shape=(jax.ShapeDtypeStruct((N, hid, HW), jnp.bfloat16),
                   jax.ShapeDtypeStruct((N, 1, hid), jnp.float32)),
        grid=(N,),
        in_specs=[
            pl.BlockSpec((1, C, HW), lambda n: (n, 0, 0)),
            pl.BlockSpec((1, 1, HW), lambda n: (n, 0, 0)),
            pl.BlockSpec((hid, C), lambda n: (0, 0)),
            pl.BlockSpec((3, hid, 1), lambda n: (0, 0, 0)),
            pl.BlockSpec((hid, 9 * hid), lambda n: (0, 0)),
            pl.BlockSpec((hid, 9 * hid), lambda n: (0, 0)),
        ],
        out_specs=(
            pl.BlockSpec((1, hid, HW), lambda n: (n, 0, 0)),
            pl.BlockSpec((1, 1, hid), lambda n: (n, 0, 0)),
        ),
        compiler_params=pltpu.CompilerParams(dimension_semantics=("parallel",)),
    )(x, conf, wenc_bf, bvec, w1_cat, w2_cat)


def _gat_body(pooled_ref, adj_ref, wgat_ref, usrc_ref, udst_ref, bgat_ref,
              wout_ref, c_ref, *, num_layers, heads):
    N = adj_ref.shape[0]
    hid = bgat_ref.shape[2]
    C = wout_ref.shape[0]
    neg_slope = 0.2

    adj = adj_ref[...]
    xg = pooled_ref[...].reshape(N, hid)

    for l in range(num_layers):
        h_all = jnp.dot(xg, wgat_ref[l], preferred_element_type=jnp.float32)
        s_all = jax.lax.dot_general(usrc_ref[l], xg, (((0,), (1,)), ((), ())),
                                    preferred_element_type=jnp.float32)
        d_all = jnp.dot(xg, udst_ref[l], preferred_element_type=jnp.float32)
        acc = jnp.zeros((N, hid), jnp.float32)
        for hd in range(heads):
            e = d_all[:, hd:hd + 1] + s_all[hd:hd + 1, :]
            e = jnp.where(e > 0, e, neg_slope * e)
            e = jnp.where(adj > 0, e, -1e9)
            e = e - jnp.max(e, axis=-1, keepdims=True)
            pr = jnp.exp(e)
            pr = pr / jnp.sum(pr, axis=-1, keepdims=True)
            acc = acc + jnp.dot(pr, h_all[:, hd * hid:(hd + 1) * hid],
                                preferred_element_type=jnp.float32)
        xg = jnp.maximum(acc * (1.0 / heads) + bgat_ref[l], 0.0)

    zt = jax.lax.dot_general(xg, wout_ref[...], (((1,), (1,)), ((), ())),
                             preferred_element_type=jnp.float32)
    c_ref[...] = zt.reshape(N, 1, C)


def _run_gat(pooled, adj, w_gat, u_src, u_dst, b_gat, w_out,
             num_layers, heads):
    N = adj.shape[0]
    hid = pooled.shape[2]
    C = w_out.shape[0]
    body = functools.partial(_gat_body, num_layers=num_layers, heads=heads)
    return pl.pallas_call(
        body,
        out_shape=jax.ShapeDtypeStruct((N, 1, C), jnp.float32),
        grid=(1,),
        in_specs=[
            pl.BlockSpec((N, 1, hid), lambda i: (0, 0, 0)),
            pl.BlockSpec((N, N), lambda i: (0, 0)),
            pl.BlockSpec((num_layers, hid, heads * hid), lambda i: (0, 0, 0)),
            pl.BlockSpec((num_layers, hid, heads), lambda i: (0, 0, 0)),
            pl.BlockSpec((num_layers, hid, heads), lambda i: (0, 0, 0)),
            pl.BlockSpec((num_layers, 1, hid), lambda i: (0, 0, 0)),
            pl.BlockSpec((C, hid), lambda i: (0, 0)),
        ],
        out_specs=pl.BlockSpec((N, 1, C), lambda i: (0, 0, 0)),
        compiler_params=pltpu.CompilerParams(dimension_semantics=("arbitrary",)),
    )(pooled, adj, w_gat, u_src, u_dst, b_gat, w_out)


def _combine_body(h_ref, c_ref, wout_ref, bout_ref, out_ref):
    C = wout_ref.shape[0]
    HW = h_ref.shape[2]
    y = jnp.dot(wout_ref[...], h_ref[0], preferred_element_type=jnp.float32)
    cn = c_ref[0]
    ones = jnp.full((1, HW), 1.0, jnp.float32)
    corr = jax.lax.dot_general(cn, ones, (((0,), (0,)), ((), ())),
                               preferred_element_type=jnp.float32)
    out_ref[...] = (y + corr + bout_ref[...]).reshape(1, C, HW)


def _run_combine(hproc, cvec, wout_bf, b_out):
    N, hid, HW = hproc.shape
    C = wout_bf.shape[0]
    return pl.pallas_call(
        _combine_body,
        out_shape=jax.ShapeDtypeStruct((N, C, HW), jnp.float32),
        grid=(N,),
        in_specs=[
            pl.BlockSpec((1, hid, HW), lambda n: (n, 0, 0)),
            pl.BlockSpec((1, 1, C), lambda n: (n, 0, 0)),
            pl.BlockSpec((C, hid), lambda n: (0, 0)),
            pl.BlockSpec((C, 1), lambda n: (0, 0)),
        ],
        out_specs=pl.BlockSpec((1, C, HW), lambda n: (n, 0, 0)),
        compiler_params=pltpu.CompilerParams(dimension_semantics=("parallel",)),
    )(hproc, cvec, wout_bf, b_out)


def kernel(x, edge_index, confidence_maps, w_enc, bvec, w_sp1, w_sp2,
           w_gat, u_src, u_dst, b_gat, w_out, b_out):
    N, C, H, W = x.shape
    HW = H * W
    hid = w_enc.shape[0]
    num_layers = w_gat.shape[0]
    heads = u_src.shape[2]

    adj = jnp.zeros((N, N), jnp.float32)
    adj = adj.at[edge_index[1], edge_index[0]].set(1.0)
    adj = adj.at[jnp.arange(N), jnp.arange(N)].set(1.0)

    x_flat = x.reshape(N, C, HW)
    conf_flat = confidence_maps.reshape(N, 1, HW)

    wenc_bf = w_enc.astype(jnp.bfloat16)
    w1_cat = jnp.transpose(w_sp1, (1, 0, 2)).reshape(hid, 9 * hid).astype(jnp.bfloat16)
    w2_cat = jnp.transpose(w_sp2, (1, 0, 2)).reshape(hid, 9 * hid).astype(jnp.bfloat16)
    wout_bf = w_out.astype(jnp.bfloat16)

    hproc, pooled = _run_spatial(x_flat, conf_flat, wenc_bf, bvec,
                                 w1_cat, w2_cat, H, W)
    cvec = _run_gat(pooled, adj, w_gat, u_src, u_dst, b_gat, w_out,
                    num_layers, heads)
    out = _run_combine(hproc, cvec, wout_bf, b_out)
    return out.reshape(N, C, H, W)


# speedup vs baseline: 1.3071x; 1.0633x over previous
import functools

import jax
import jax.numpy as jnp
from jax.experimental import pallas as pl
from jax.experimental.pallas import tpu as pltpu


def _spatial_body(x_ref, conf_ref, wenc_ref, bvec_ref, w1_ref, w2_ref,
                  hout_ref, pooled_ref, *, H, W):
    HW = H * W
    hid = wenc_ref.shape[0]

    x = x_ref[0].astype(jnp.bfloat16)
    conf = conf_ref[0]

    h = jnp.dot(wenc_ref[...], x, preferred_element_type=jnp.float32)
    h = jnp.maximum(h + bvec_ref[0], 0.0) * conf

    p = jax.lax.broadcasted_iota(jnp.int32, (1, HW), 1)
    xcol = p % W
    yrow = p // W
    m_xm = xcol >= 1
    m_xp = xcol <= W - 2
    m_ym = yrow >= 1
    m_yp = yrow <= H - 2

    def conv3x3(v, w_ref, bias):
        zero = jnp.zeros((), jnp.bfloat16)
        vxm = jnp.where(m_xm, pltpu.roll(v, 1, axis=1), zero)
        vxp = jnp.where(m_xp, pltpu.roll(v, HW - 1, axis=1), zero)
        c3 = jnp.concatenate([vxm, v, vxp], axis=0)
        up = jnp.where(m_ym, pltpu.roll(c3, W, axis=1), zero)
        dn = jnp.where(m_yp, pltpu.roll(c3, HW - W, axis=1), zero)
        u = jnp.concatenate([up, c3, dn], axis=0)
        g = jnp.dot(w_ref[...], u, preferred_element_type=jnp.float32)
        return jnp.maximum(g + bias, 0.0)

    h1 = conv3x3(h.astype(jnp.bfloat16), w1_ref, bvec_ref[1])
    h2 = conv3x3(h1.astype(jnp.bfloat16), w2_ref, bvec_ref[2])

    hout_ref[...] = h2.astype(jnp.bfloat16).reshape(1, hid, HW)

    inv_hw = jnp.full((1, HW), 1.0 / HW, jnp.float32)
    pooled = jax.lax.dot_general(inv_hw, h2, (((1,), (1,)), ((), ())),
                                 preferred_element_type=jnp.float32)
    pooled_ref[...] = pooled.reshape(1, 1, hid)


def _run_spatial(x, conf, wenc_bf, bvec, w1_cat, w2_cat, H, W):
    N, C, HW = x.shape
    hid = wenc_bf.shape[0]
    body = functools.partial(_spatial_body, H=H, W=W)
    return pl.pallas_call(
        body,
        out_shape=(jax.ShapeDtypeStruct((N, hid, HW), jnp.bfloat16),
                   jax.ShapeDtypeStruct((N, 1, hid), jnp.float32)),
        grid=(N,),
        in_specs=[
            pl.BlockSpec((1, C, HW), lambda n: (n, 0, 0)),
            pl.BlockSpec((1, 1, HW), lambda n: (n, 0, 0)),
            pl.BlockSpec((hid, C), lambda n: (0, 0)),
            pl.BlockSpec((3, hid, 1), lambda n: (0, 0, 0)),
            pl.BlockSpec((hid, 9 * hid), lambda n: (0, 0)),
            pl.BlockSpec((hid, 9 * hid), lambda n: (0, 0)),
        ],
        out_specs=(
            pl.BlockSpec((1, hid, HW), lambda n: (n, 0, 0)),
            pl.BlockSpec((1, 1, hid), lambda n: (n, 0, 0)),
        ),
        compiler_params=pltpu.CompilerParams(dimension_semantics=("parallel",)),
    )(x, conf, wenc_bf, bvec, w1_cat, w2_cat)


def _gat_body(pooled_ref, ei_ref, wgat_ref, usrc_ref, udst_ref, bgat_ref,
              wout_ref, c_ref, *, num_layers, heads):
    N = pooled_ref.shape[0]
    hid = bgat_ref.shape[2]
    C = wout_ref.shape[0]
    E = ei_ref.shape[1]
    neg_slope = 0.2

    ii = jax.lax.broadcasted_iota(jnp.int32, (N, E), 0)
    don = (ii == ei_ref[1:2, :]).astype(jnp.float32)
    son = (ii == ei_ref[0:1, :]).astype(jnp.float32)
    cnt = jax.lax.dot_general(don, son, (((1,), (1,)), ((), ())),
                              preferred_element_type=jnp.float32)
    ri = jax.lax.broadcasted_iota(jnp.int32, (N, N), 0)
    ci = jax.lax.broadcasted_iota(jnp.int32, (N, N), 1)
    adj = jnp.logical_or(cnt > 0, ri == ci)

    xg = pooled_ref[...].reshape(N, hid)

    for l in range(num_layers):
        h_all = jnp.dot(xg, wgat_ref[l], preferred_element_type=jnp.float32)
        s_all = jax.lax.dot_general(usrc_ref[l], xg, (((0,), (1,)), ((), ())),
                                    preferred_element_type=jnp.float32)
        d_all = jnp.dot(xg, udst_ref[l], preferred_element_type=jnp.float32)
        acc = jnp.zeros((N, hid), jnp.float32)
        for hd in range(heads):
            e = d_all[:, hd:hd + 1] + s_all[hd:hd + 1, :]
            e = jnp.where(e > 0, e, neg_slope * e)
            e = jnp.where(adj, e, -1e9)
            e = e - jnp.max(e, axis=-1, keepdims=True)
            pr = jnp.exp(e)
            pr = pr / jnp.sum(pr, axis=-1, keepdims=True)
            acc = acc + jnp.dot(pr, h_all[:, hd * hid:(hd + 1) * hid],
                                preferred_element_type=jnp.float32)
        xg = jnp.maximum(acc * (1.0 / heads) + bgat_ref[l], 0.0)

    zt = jax.lax.dot_general(xg, wout_ref[...], (((1,), (1,)), ((), ())),
                             preferred_element_type=jnp.float32)
    c_ref[...] = zt.reshape(N, 1, C)


def _run_gat(pooled, edge_index, w_gat, u_src, u_dst, b_gat, w_out,
             num_layers, heads):
    N = pooled.shape[0]
    hid = pooled.shape[2]
    C = w_out.shape[0]
    E = edge_index.shape[1]
    body = functools.partial(_gat_body, num_layers=num_layers, heads=heads)
    return pl.pallas_call(
        body,
        out_shape=jax.ShapeDtypeStruct((N, 1, C), jnp.float32),
        grid=(1,),
        in_specs=[
            pl.BlockSpec((N, 1, hid), lambda i: (0, 0, 0)),
            pl.BlockSpec((2, E), lambda i: (0, 0)),
            pl.BlockSpec((num_layers, hid, heads * hid), lambda i: (0, 0, 0)),
            pl.BlockSpec((num_layers, hid, heads), lambda i: (0, 0, 0)),
            pl.BlockSpec((num_layers, hid, heads), lambda i: (0, 0, 0)),
            pl.BlockSpec((num_layers, 1, hid), lambda i: (0, 0, 0)),
            pl.BlockSpec((C, hid), lambda i: (0, 0)),
        ],
        out_specs=pl.BlockSpec((N, 1, C), lambda i: (0, 0, 0)),
        compiler_params=pltpu.CompilerParams(dimension_semantics=("arbitrary",)),
    )(pooled, edge_index, w_gat, u_src, u_dst, b_gat, w_out)


def _combine_body(h_ref, c_ref, wout_ref, bout_ref, out_ref):
    C = wout_ref.shape[0]
    HW = h_ref.shape[2]
    y = jnp.dot(wout_ref[...], h_ref[0], preferred_element_type=jnp.float32)
    cn = c_ref[0]
    ones = jnp.full((1, HW), 1.0, jnp.float32)
    corr = jax.lax.dot_general(cn, ones, (((0,), (0,)), ((), ())),
                               preferred_element_type=jnp.float32)
    out_ref[...] = (y + corr + bout_ref[...]).reshape(1, C, HW)


def _run_combine(hproc, cvec, wout_bf, b_out):
    N, hid, HW = hproc.shape
    C = wout_bf.shape[0]
    return pl.pallas_call(
        _combine_body,
        out_shape=jax.ShapeDtypeStruct((N, C, HW), jnp.float32),
        grid=(N,),
        in_specs=[
            pl.BlockSpec((1, hid, HW), lambda n: (n, 0, 0)),
            pl.BlockSpec((1, 1, C), lambda n: (n, 0, 0)),
            pl.BlockSpec((C, hid), lambda n: (0, 0)),
            pl.BlockSpec((C, 1), lambda n: (0, 0)),
        ],
        out_specs=pl.BlockSpec((1, C, HW), lambda n: (n, 0, 0)),
        compiler_params=pltpu.CompilerParams(dimension_semantics=("parallel",)),
    )(hproc, cvec, wout_bf, b_out)


def kernel(x, edge_index, confidence_maps, w_enc, bvec, w_sp1, w_sp2,
           w_gat, u_src, u_dst, b_gat, w_out, b_out):
    N, C, H, W = x.shape
    HW = H * W
    hid = w_enc.shape[0]
    num_layers = w_gat.shape[0]
    heads = u_src.shape[2]

    x_flat = x.reshape(N, C, HW)
    conf_flat = confidence_maps.reshape(N, 1, HW)

    wenc_bf = w_enc.astype(jnp.bfloat16)
    w1_cat = jnp.transpose(w_sp1, (1, 0, 2)).reshape(hid, 9 * hid).astype(jnp.bfloat16)
    w2_cat = jnp.transpose(w_sp2, (1, 0, 2)).reshape(hid, 9 * hid).astype(jnp.bfloat16)
    wout_bf = w_out.astype(jnp.bfloat16)

    hproc, pooled = _run_spatial(x_flat, conf_flat, wenc_bf, bvec,
                                 w1_cat, w2_cat, H, W)
    cvec = _run_gat(pooled, edge_index, w_gat, u_src, u_dst, b_gat, w_out,
                    num_layers, heads)
    out = _run_combine(hproc, cvec, wout_bf, b_out)
    return out.reshape(N, C, H, W)
